# baseline (speedup 1.0000x reference)
import functools
import jax
import jax.numpy as jnp
from jax import lax
from jax.experimental import pallas as pl
from jax.experimental.pallas import tpu as pltpu
from jax.experimental.pallas import tpu_sc as plsc

_MAX_OUT = 200
_N = 5000
_LANES = 16
_SUBCORES = 16
_NP = 5120
_PER = _NP // _SUBCORES
_VPT = _PER // _LANES
_ROW = 8
_PLANE = _SUBCORES * _ROW
_SH = 2 * 2 * _PLANE

_NEG = float("-inf")


def _nms_body(bx_h, by_h, bw_h, bh_h, sc_h, out_h,
              bxv, byv, bwv, bhv,
              x1f, x3f, y1f, y3f, arf,
              msc, sel, outv,
              pub, allb, sh):
    c = lax.axis_index("c")
    s = lax.axis_index("s")
    base = s * _PER

    pltpu.sync_copy(bx_h, bxv)
    pltpu.sync_copy(by_h, byv)
    pltpu.sync_copy(bw_h, bwv)
    pltpu.sync_copy(bh_h, bhv)
    pltpu.sync_copy(sc_h.at[pl.ds(base, _PER)], msc)

    def initj(j, _):
        sl = pl.ds(j * _LANES, _LANES)
        x = bxv[sl]
        y = byv[sl]
        w = bwv[sl]
        h = bhv[sl]
        x1f[sl] = x - 0.5 * w
        x3f[sl] = x + 0.5 * w
        y1f[sl] = y - 0.5 * h
        y3f[sl] = y + 0.5 * h
        arf[sl] = w * h
        return 0

    lax.fori_loop(0, _NP // _LANES, initj, 0)

    zeros16 = jnp.zeros((_LANES,), jnp.float32)
    for j in range(_VPT):
        sel[pl.ds(j * _LANES, _LANES)] = zeros16

    iota16 = lax.iota(jnp.int32, _LANES)
    negv = jnp.full((_LANES,), _NEG, jnp.float32)
    ones16 = jnp.full((_LANES,), 1.0, jnp.float32)
    izeros16 = jnp.zeros((_LANES,), jnp.int32)
    bigv = jnp.full((_LANES,), 2 ** 30, jnp.int32)
    basev = jnp.full((_LANES,), base, jnp.int32)
    perv = jnp.full((_LANES,), _PER, jnp.int32)
    lane0 = iota16 == izeros16
    lane1 = iota16 == jnp.full((_LANES,), 1, jnp.int32)
    cplane = c * _PLANE

    def xmax(v):
        for shf in (8, 4, 2, 1):
            v = jnp.maximum(v, v.at[iota16 ^ shf].get(mode="promise_in_bounds"))
        return v

    def xmin_i(v):
        for shf in (8, 4, 2, 1):
            v = jnp.minimum(v, v.at[iota16 ^ shf].get(mode="promise_in_bounds"))
        return v

    def one_round(par, bv, bi, curs):
        mx = xmax(bv)
        mi = xmin_i(jnp.where(bv == mx, bi, bigv))

        mi_pub = jnp.where(mx > negv, mi, izeros16)
        pub[...] = jnp.where(lane1, plsc.bitcast(mi_pub, jnp.float32), mx)
        plane = cplane + par * 2 * _PLANE
        pltpu.sync_copy(pub.at[pl.ds(0, _ROW)], sh.at[pl.ds(plane + s * _ROW, _ROW)])
        plsc.subcore_barrier()
        pltpu.sync_copy(sh.at[pl.ds(plane, _PLANE)], allb)

        gv = plsc.load_gather(allb, [iota16 * _ROW])
        gi = plsc.bitcast(plsc.load_gather(allb, [iota16 * _ROW + jnp.full((_LANES,), 1, jnp.int32)]), jnp.int32)
        cx1 = plsc.load_gather(x1f, [gi])
        cx3 = plsc.load_gather(x3f, [gi])
        cy1 = plsc.load_gather(y1f, [gi])
        cy3 = plsc.load_gather(y3f, [gi])
        car = plsc.load_gather(arf, [gi])
        gm = xmax(gv)
        f = plsc.all_reduce_ffs(gv == gm)
        win = gi.at[f].get(mode="promise_in_bounds")
        valid = gm > negv

        gx1 = cx1.at[f].get(mode="promise_in_bounds")
        gx3 = cx3.at[f].get(mode="promise_in_bounds")
        gy1 = cy1.at[f].get(mode="promise_in_bounds")
        gy3 = cy3.at[f].get(mode="promise_in_bounds")
        gar = car.at[f].get(mode="promise_in_bounds")

        lidx = win - basev
        mine = valid & (lidx >= izeros16) & (lidx < perv) & lane0
        plsc.addupdate_scatter(sel, [jnp.where(mine, lidx, izeros16)], ones16, mask=mine)

        nbv = negv
        nbi = bigv
        ncurs = []
        for j in range(_VPT):
            fsl = pl.ds(base + j * _LANES, _LANES)
            xx1 = jnp.maximum(x1f[fsl], gx1)
            xx3 = jnp.minimum(x3f[fsl], gx3)
            yy1 = jnp.maximum(y1f[fsl], gy1)
            yy3 = jnp.minimum(y3f[fsl], gy3)
            inter = jnp.maximum(xx3 - xx1, 0.0) * jnp.maximum(yy3 - yy1, 0.0)
            union = arf[fsl] + gar - inter
            supp = ((33554432.0 * inter - 16777216.0 * union) > union) & valid
            nc = jnp.where(supp, negv, curs[j])
            ncurs.append(nc)
            gio = iota16 + jnp.full((_LANES,), j * _LANES, jnp.int32) + basev
            better = nc > nbv
            nbv = jnp.where(better, nc, nbv)
            nbi = jnp.where(better, gio, nbi)
        return nbv, nbi, ncurs

    def round_pair(i, carry):
        bv, bi = carry[0], carry[1]
        curs = list(carry[2:])
        bv, bi, curs = one_round(0, bv, bi, curs)
        bv, bi, curs = one_round(1, bv, bi, curs)
        return (bv, bi) + tuple(curs)

    curs0 = [msc[pl.ds(j * _LANES, _LANES)] for j in range(_VPT)]
    bv0 = curs0[0]
    bi0 = iota16 + basev
    for j in range(1, _VPT):
        v = curs0[j]
        gio0 = iota16 + jnp.full((_LANES,), j * _LANES, jnp.int32) + basev
        better0 = v > bv0
        bv0 = jnp.where(better0, v, bv0)
        bi0 = jnp.where(better0, gio0, bi0)
    lax.fori_loop(0, _MAX_OUT // 2, round_pair, (bv0, bi0) + tuple(curs0))

    for j in range(_VPT):
        sl = pl.ds(j * _LANES, _LANES)
        outv[sl] = msc[sl] * sel[sl]

    @pl.when(c == 0)
    def _():
        pltpu.sync_copy(outv, out_h.at[pl.ds(base, _PER)])


@jax.jit
def _nms_sc(bx, by, bw, bh, sc):
    mesh = plsc.VectorSubcoreMesh(core_axis_name="c", subcore_axis_name="s")
    f = functools.partial(
        pl.kernel,
        mesh=mesh,
        compiler_params=pltpu.CompilerParams(needs_layout_passes=False),
        out_type=jax.ShapeDtypeStruct((_NP,), jnp.float32),
        scratch_types=[
            pltpu.VMEM((_NP,), jnp.float32),
            pltpu.VMEM((_NP,), jnp.float32),
            pltpu.VMEM((_NP,), jnp.float32),
            pltpu.VMEM((_NP,), jnp.float32),
            pltpu.VMEM((_NP,), jnp.float32),
            pltpu.VMEM((_NP,), jnp.float32),
            pltpu.VMEM((_NP,), jnp.float32),
            pltpu.VMEM((_NP,), jnp.float32),
            pltpu.VMEM((_NP,), jnp.float32),
            pltpu.VMEM((_PER,), jnp.float32),
            pltpu.VMEM((_PER,), jnp.float32),
            pltpu.VMEM((_PER,), jnp.float32),
            pltpu.VMEM((_LANES,), jnp.float32),
            pltpu.VMEM((_PLANE,), jnp.float32),
            pltpu.VMEM_SHARED((_SH,), jnp.float32),
        ],
    )(_nms_body)
    return f(bx, by, bw, bh, sc)


def kernel(boxes, scores):
    pad = _NP - _N
    bx = jnp.concatenate([boxes[:, 0], jnp.zeros((pad,), jnp.float32)])
    by = jnp.concatenate([boxes[:, 1], jnp.zeros((pad,), jnp.float32)])
    bw = jnp.concatenate([boxes[:, 2], jnp.zeros((pad,), jnp.float32)])
    bh = jnp.concatenate([boxes[:, 3], jnp.zeros((pad,), jnp.float32)])
    sc = jnp.concatenate([scores, jnp.full((pad,), _NEG, jnp.float32)])
    out = _nms_sc(bx, by, bw, bh, sc)
    return out[:_N]


# speedup vs baseline: 1.0049x; 1.0049x over previous
import functools
import jax
import jax.numpy as jnp
from jax import lax
from jax.experimental import pallas as pl
from jax.experimental.pallas import tpu as pltpu
from jax.experimental.pallas import tpu_sc as plsc

_MAX_OUT = 200
_N = 5000
_LANES = 16
_SUBCORES = 16
_NP = 5120
_PER = _NP // _SUBCORES
_VPT = _PER // _LANES
_ROW = 8
_PLANE = _SUBCORES * _ROW
_SH = 2 * 2 * _PLANE

_NEG = float("-inf")


def _nms_body(bt_h, sc_h, out_h,
              bxv, byv, bwv, bhv,
              x1f, x3f, y1f, y3f, arf,
              msc, sel, outv,
              pub, allb, sh):
    c = lax.axis_index("c")
    s = lax.axis_index("s")
    base = s * _PER

    pltpu.sync_copy(bt_h.at[0], bxv)
    pltpu.sync_copy(bt_h.at[1], byv)
    pltpu.sync_copy(bt_h.at[2], bwv)
    pltpu.sync_copy(bt_h.at[3], bhv)
    pltpu.sync_copy(sc_h.at[pl.ds(base, _PER)], msc)

    def initj(j, _):
        sl = pl.ds(j * _LANES, _LANES)
        x = bxv[sl]
        y = byv[sl]
        w = bwv[sl]
        h = bhv[sl]
        x1f[sl] = x - 0.5 * w
        x3f[sl] = x + 0.5 * w
        y1f[sl] = y - 0.5 * h
        y3f[sl] = y + 0.5 * h
        arf[sl] = w * h
        return 0

    lax.fori_loop(0, _NP // _LANES, initj, 0)

    zeros16 = jnp.zeros((_LANES,), jnp.float32)
    for j in range(_VPT):
        sel[pl.ds(j * _LANES, _LANES)] = zeros16

    iota16 = lax.iota(jnp.int32, _LANES)
    negv = jnp.full((_LANES,), _NEG, jnp.float32)
    ones16 = jnp.full((_LANES,), 1.0, jnp.float32)
    izeros16 = jnp.zeros((_LANES,), jnp.int32)
    bigv = jnp.full((_LANES,), 2 ** 30, jnp.int32)
    basev = jnp.full((_LANES,), base, jnp.int32)
    perv = jnp.full((_LANES,), _PER, jnp.int32)
    lane0 = iota16 == izeros16
    lane1 = iota16 == jnp.full((_LANES,), 1, jnp.int32)
    cplane = c * _PLANE

    def xmax(v):
        for shf in (8, 4, 2, 1):
            v = jnp.maximum(v, v.at[iota16 ^ shf].get(mode="promise_in_bounds"))
        return v

    def xmin_i(v):
        for shf in (8, 4, 2, 1):
            v = jnp.minimum(v, v.at[iota16 ^ shf].get(mode="promise_in_bounds"))
        return v

    def one_round(par, bv, bi, curs):
        mx = xmax(bv)
        mi = xmin_i(jnp.where(bv == mx, bi, bigv))

        mi_pub = jnp.where(mx > negv, mi, izeros16)
        pub[...] = jnp.where(lane1, plsc.bitcast(mi_pub, jnp.float32), mx)
        plane = cplane + par * 2 * _PLANE
        pltpu.sync_copy(pub.at[pl.ds(0, _ROW)], sh.at[pl.ds(plane + s * _ROW, _ROW)])
        plsc.subcore_barrier()
        pltpu.sync_copy(sh.at[pl.ds(plane, _PLANE)], allb)

        gv = plsc.load_gather(allb, [iota16 * _ROW])
        gi = plsc.bitcast(plsc.load_gather(allb, [iota16 * _ROW + jnp.full((_LANES,), 1, jnp.int32)]), jnp.int32)
        cx1 = plsc.load_gather(x1f, [gi])
        cx3 = plsc.load_gather(x3f, [gi])
        cy1 = plsc.load_gather(y1f, [gi])
        cy3 = plsc.load_gather(y3f, [gi])
        car = plsc.load_gather(arf, [gi])
        gm = xmax(gv)
        f = plsc.all_reduce_ffs(gv == gm)
        win = gi.at[f].get(mode="promise_in_bounds")
        valid = gm > negv

        gx1 = cx1.at[f].get(mode="promise_in_bounds")
        gx3 = cx3.at[f].get(mode="promise_in_bounds")
        gy1 = cy1.at[f].get(mode="promise_in_bounds")
        gy3 = cy3.at[f].get(mode="promise_in_bounds")
        gar = car.at[f].get(mode="promise_in_bounds")

        lidx = win - basev
        mine = valid & (lidx >= izeros16) & (lidx < perv) & lane0
        plsc.addupdate_scatter(sel, [jnp.where(mine, lidx, izeros16)], ones16, mask=mine)

        nbv = negv
        nbi = bigv
        ncurs = []
        for j in range(_VPT):
            fsl = pl.ds(base + j * _LANES, _LANES)
            xx1 = jnp.maximum(x1f[fsl], gx1)
            xx3 = jnp.minimum(x3f[fsl], gx3)
            yy1 = jnp.maximum(y1f[fsl], gy1)
            yy3 = jnp.minimum(y3f[fsl], gy3)
            inter = jnp.maximum(xx3 - xx1, 0.0) * jnp.maximum(yy3 - yy1, 0.0)
            union = arf[fsl] + gar - inter
            supp = ((33554432.0 * inter - 16777216.0 * union) > union) & valid
            nc = jnp.where(supp, negv, curs[j])
            ncurs.append(nc)
            gio = iota16 + jnp.full((_LANES,), j * _LANES, jnp.int32) + basev
            better = nc > nbv
            nbv = jnp.where(better, nc, nbv)
            nbi = jnp.where(better, gio, nbi)
        return nbv, nbi, ncurs

    def round_pair(i, carry):
        bv, bi = carry[0], carry[1]
        curs = list(carry[2:])
        bv, bi, curs = one_round(0, bv, bi, curs)
        bv, bi, curs = one_round(1, bv, bi, curs)
        return (bv, bi) + tuple(curs)

    curs0 = [msc[pl.ds(j * _LANES, _LANES)] for j in range(_VPT)]
    bv0 = curs0[0]
    bi0 = iota16 + basev
    for j in range(1, _VPT):
        v = curs0[j]
        gio0 = iota16 + jnp.full((_LANES,), j * _LANES, jnp.int32) + basev
        better0 = v > bv0
        bv0 = jnp.where(better0, v, bv0)
        bi0 = jnp.where(better0, gio0, bi0)
    lax.fori_loop(0, _MAX_OUT // 2, round_pair, (bv0, bi0) + tuple(curs0))

    for j in range(_VPT):
        sl = pl.ds(j * _LANES, _LANES)
        outv[sl] = msc[sl] * sel[sl]

    @pl.when(c == 0)
    def _():
        pltpu.sync_copy(outv, out_h.at[pl.ds(base, _PER)])


@jax.jit
def _nms_sc(bt, sc):
    mesh = plsc.VectorSubcoreMesh(core_axis_name="c", subcore_axis_name="s")
    f = functools.partial(
        pl.kernel,
        mesh=mesh,
        compiler_params=pltpu.CompilerParams(needs_layout_passes=False),
        out_type=jax.ShapeDtypeStruct((_NP,), jnp.float32),
        scratch_types=[
            pltpu.VMEM((_NP,), jnp.float32),
            pltpu.VMEM((_NP,), jnp.float32),
            pltpu.VMEM((_NP,), jnp.float32),
            pltpu.VMEM((_NP,), jnp.float32),
            pltpu.VMEM((_NP,), jnp.float32),
            pltpu.VMEM((_NP,), jnp.float32),
            pltpu.VMEM((_NP,), jnp.float32),
            pltpu.VMEM((_NP,), jnp.float32),
            pltpu.VMEM((_NP,), jnp.float32),
            pltpu.VMEM((_PER,), jnp.float32),
            pltpu.VMEM((_PER,), jnp.float32),
            pltpu.VMEM((_PER,), jnp.float32),
            pltpu.VMEM((_LANES,), jnp.float32),
            pltpu.VMEM((_PLANE,), jnp.float32),
            pltpu.VMEM_SHARED((_SH,), jnp.float32),
        ],
    )(_nms_body)
    return f(bt, sc)


def kernel(boxes, scores):
    bt = jnp.zeros((4, _NP), jnp.float32).at[:, :_N].set(boxes.T)
    sc = jnp.full((_NP,), _NEG, jnp.float32).at[:_N].set(scores)
    out = _nms_sc(bt, sc)
    return out[:_N]
